# baseline (speedup 1.0000x reference)
import functools

import jax
import jax.numpy as jnp
from jax import lax
from jax.experimental import pallas as pl
from jax.experimental.pallas import tpu as pltpu
from jax.experimental.pallas import tpu_sc as plsc

_ND = 4000
_NR = 6000
_N = 10000
_D = 128
_E = 256000
_NC = 2
_NS = 16
_B = 4096

_C = 64
_P = _C // 2
_EPT = _E // _NS
_NCHUNK = _EPT // _C
_NPAIRIT = _NCHUNK // 2
_IW = 6 * _P

_RPT = 624
_RC = 104
_RTAIL = _N - _NS * _RPT


def _gate_body(emb_ref, w_ref, b_ref, out_ref):
    e = emb_ref[...]
    z = jnp.dot(e, w_ref[...], preferred_element_type=jnp.float32) + b_ref[...]
    out_ref[...] = e * jax.nn.sigmoid(z)


def _gate(emb, W, b):
    return pl.pallas_call(
        _gate_body,
        out_shape=jax.ShapeDtypeStruct(emb.shape, jnp.float32),
    )(emb, W, b)


def _prop_mesh():
    return plsc.VectorSubcoreMesh(core_axis_name="c", subcore_axis_name="s")


def _hsum16(p):
    return jnp.sum(p)


@functools.partial(
    pl.kernel,
    out_type=[
        jax.ShapeDtypeStruct((2 * _N, _D), jnp.float32),
        jax.ShapeDtypeStruct((2 * _N, _D), jnp.float32),
    ],
    mesh=_prop_mesh(),
    compiler_params=pltpu.CompilerParams(needs_layout_passes=False),
    scratch_types=[
        pltpu.VMEM_SHARED((_N, _D), jnp.float32),
        pltpu.VMEM((_C, _D), jnp.float32),
        pltpu.VMEM((_C, _D), jnp.float32),
        pltpu.VMEM((_C, _D), jnp.float32),
        pltpu.VMEM((_C, _D), jnp.float32),
        pltpu.VMEM((_RC, _D), jnp.float32),
        pltpu.VMEM((_IW,), jnp.int32),
        pltpu.VMEM((_IW,), jnp.int32),
        pltpu.VMEM((_C,), jnp.int32),
        pltpu.VMEM((_C,), jnp.int32),
        pltpu.SMEM((_P,), jnp.float32),
        pltpu.SMEM((_C,), jnp.float32),
        pltpu.SemaphoreType.DMA,
        pltpu.SemaphoreType.DMA,
        pltpu.SemaphoreType.DMA,
        pltpu.SemaphoreType.DMA,
        pltpu.SemaphoreType.DMA,
        pltpu.SemaphoreType.DMA,
    ],
)
def _prop2(e0_ref, comb_ref, e1_ref, e2_ref,
           acc, gbufA, gbufB, wbufA, wbufB, dbuf, idxA, idxB, isccA, isccB,
           dsm, msm, gsA, gsB, ssA, ssB, isA, isB):
    c = lax.axis_index("c")
    s = lax.axis_index("s")
    cbase = c * (_E // _C) + s * _NCHUNK

    def _zero_dbuf():
        def _zrow(i, _):
            for v in range(_D // 16):
                dbuf[i, pl.ds(16 * v, 16)] = jnp.zeros((16,), jnp.float32)
            return 0
        lax.fori_loop(0, _RC, _zrow, 0)

    def _acc_zero():
        for r in range(_RPT // _RC):
            pltpu.sync_copy(dbuf,
                            acc.at[pl.ds(s * _RPT + r * _RC, _RC), :])

        @pl.when(s == 0)
        def _():
            pltpu.sync_copy(dbuf.at[pl.ds(0, _RTAIL), :],
                            acc.at[pl.ds(_NS * _RPT, _RTAIL), :])

    _zero_dbuf()
    _acc_zero()
    plsc.subcore_barrier()

    def _stage_iscc(idx, iscc):
        for v in range(_C // 16):
            iscc[pl.ds(16 * v, 16)] = idx[pl.ds(2 * _P + 16 * v, 16)]

    def _compute(gbuf, idx, wbuf):
        @plsc.parallel_loop(0, _P, unroll=2)
        def _dot(e):
            p01 = (gbuf[e, pl.ds(0, 16)] * gbuf[_P + e, pl.ds(0, 16)]
                   + gbuf[e, pl.ds(16, 16)] * gbuf[_P + e, pl.ds(16, 16)])
            p23 = (gbuf[e, pl.ds(32, 16)] * gbuf[_P + e, pl.ds(32, 16)]
                   + gbuf[e, pl.ds(48, 16)] * gbuf[_P + e, pl.ds(48, 16)])
            p45 = (gbuf[e, pl.ds(64, 16)] * gbuf[_P + e, pl.ds(64, 16)]
                   + gbuf[e, pl.ds(80, 16)] * gbuf[_P + e, pl.ds(80, 16)])
            p67 = (gbuf[e, pl.ds(96, 16)] * gbuf[_P + e, pl.ds(96, 16)]
                   + gbuf[e, pl.ds(112, 16)] * gbuf[_P + e, pl.ds(112, 16)])
            p = (p01 + p23) + (p45 + p67)
            dsm[e] = 2.0 - _hsum16(p) * (1.0 / 64.0)

        @plsc.parallel_loop(0, _P // 16, unroll=1)
        def _mul(g):
            vf = plsc.bitcast(idx[pl.ds(4 * _P + 16 * g, 16)], jnp.float32)
            vb = plsc.bitcast(idx[pl.ds(5 * _P + 16 * g, 16)], jnp.float32)
            for jj in range(16):
                e = g * 16 + jj
                d = dsm[e]
                msm[e] = d * vf[jj]
                msm[_P + e] = d * vb[jj]

        @plsc.parallel_loop(0, _P, unroll=2)
        def _scale(e):
            mf = msm[e]
            mb = msm[_P + e]
            for v in range(8):
                sl = pl.ds(16 * v, 16)
                ra = gbuf[e, sl]
                rb = gbuf[_P + e, sl]
                wbuf[e, sl] = rb * mf
                wbuf[_P + e, sl] = ra * mb

    def _fire_gather(src_ref, idx, gbuf, sem):
        pltpu.async_copy(src_ref.at[idx.at[pl.ds(0, _C)]], gbuf, sem)

    def _wait_gather(src_ref, idx, gbuf, sem):
        pltpu.make_async_copy(
            src_ref.at[idx.at[pl.ds(0, _C)]], gbuf, sem).wait()

    def _fire_scatter(wbuf, iscc, sem):
        pltpu.async_copy(wbuf, acc.at[iscc], sem, add=True)

    def _wait_scatter(wbuf, iscc, sem):
        pltpu.make_async_copy(wbuf, acc.at[iscc], sem).wait()

    def _fire_idx(blk, idx, sem):
        pltpu.async_copy(comb_ref.at[pl.ds(blk * _IW, _IW)], idx, sem)

    def _wait_idx(idx, sem):
        pltpu.make_async_copy(comb_ref.at[pl.ds(0, _IW)], idx, sem).wait()

    def _layer(src_ref, dst_ref, rezero):
        pltpu.sync_copy(comb_ref.at[pl.ds(cbase * _IW, _IW)], idxA)
        pltpu.sync_copy(comb_ref.at[pl.ds((cbase + 1) * _IW, _IW)], idxB)
        _fire_gather(src_ref, idxA, gbufA, gsA)

        def _pair(q, _):
            j0 = 2 * q

            _wait_gather(src_ref, idxA, gbufA, gsA)

            @pl.when(q > 0)
            def _():
                _wait_scatter(wbufA, isccA, ssA)
                _wait_idx(idxB, isB)

            _stage_iscc(idxA, isccA)
            _fire_gather(src_ref, idxB, gbufB, gsB)
            _compute(gbufA, idxA, wbufA)
            _fire_scatter(wbufA, isccA, ssA)

            @pl.when(j0 + 2 < _NCHUNK)
            def _():
                _fire_idx(cbase + j0 + 2, idxA, isA)

            _wait_gather(src_ref, idxB, gbufB, gsB)

            @pl.when(q > 0)
            def _():
                _wait_scatter(wbufB, isccB, ssB)

            _stage_iscc(idxB, isccB)

            @pl.when(j0 + 2 < _NCHUNK)
            def _():
                _wait_idx(idxA, isA)
                _fire_gather(src_ref, idxA, gbufA, gsA)

            _compute(gbufB, idxB, wbufB)
            _fire_scatter(wbufB, isccB, ssB)

            @pl.when(j0 + 3 < _NCHUNK)
            def _():
                _fire_idx(cbase + j0 + 3, idxB, isB)

            return 0

        lax.fori_loop(0, _NPAIRIT, _pair, 0)
        _wait_scatter(wbufA, isccA, ssA)
        _wait_scatter(wbufB, isccB, ssB)
        plsc.subcore_barrier()

        def _dump(row0, nrows):
            pltpu.sync_copy(acc.at[pl.ds(row0, nrows), :],
                            dbuf.at[pl.ds(0, nrows), :])
            pltpu.sync_copy(dbuf.at[pl.ds(0, nrows), :],
                            dst_ref.at[pl.ds(c * _N + row0, nrows), :])

        for r in range(_RPT // _RC):
            _dump(s * _RPT + r * _RC, _RC)

        @pl.when(s == 0)
        def _():
            _dump(_NS * _RPT, _RTAIL)

        if rezero:
            _zero_dbuf()
            _acc_zero()
        plsc.subcore_barrier()

    _layer(e0_ref, e1_ref, True)
    _layer(e1_ref, e2_ref, False)


_PC = _B // (_NC * _NS)


@functools.partial(
    pl.kernel,
    out_type=jax.ShapeDtypeStruct((_B,), jnp.float32),
    mesh=_prop_mesh(),
    compiler_params=pltpu.CompilerParams(needs_layout_passes=False),
    scratch_types=[
        pltpu.VMEM((_PC,), jnp.int32),
        pltpu.VMEM((_PC,), jnp.int32),
        pltpu.VMEM((_PC,), jnp.int32),
        pltpu.VMEM((_PC,), jnp.int32),
        pltpu.VMEM((_PC, _D), jnp.float32),
        pltpu.VMEM((_PC, _D), jnp.float32),
        pltpu.VMEM((_PC, _D), jnp.float32),
        pltpu.VMEM((_PC,), jnp.float32),
        pltpu.SMEM((_PC,), jnp.float32),
        pltpu.SemaphoreType.DMA,
    ],
)
def _score(e0_ref, e1_ref, e2_ref, d1_ref, d2_ref, r1_ref, r2_ref, out_ref,
           di1, di2, ri1, ri2, accA, accB, tmp, obuf, psm, sem):
    c = lax.axis_index("c")
    s = lax.axis_index("s")
    wid = s * _NC + c
    base = wid * _PC

    pltpu.sync_copy(d1_ref.at[pl.ds(base, _PC)], di1)
    pltpu.sync_copy(d2_ref.at[pl.ds(base, _PC)], di2)
    pltpu.sync_copy(r1_ref.at[pl.ds(base, _PC)], ri1)
    pltpu.sync_copy(r2_ref.at[pl.ds(base, _PC)], ri2)

    def _accum(dst, idx_ref, table_ref, first):
        pltpu.async_copy(table_ref.at[idx_ref], tmp, sem).wait()

        def _row(e, _):
            for v in range(_D // 16):
                sl = pl.ds(16 * v, 16)
                if first:
                    dst[e, sl] = tmp[e, sl]
                else:
                    dst[e, sl] = dst[e, sl] + tmp[e, sl]
            return 0

        lax.fori_loop(0, _PC, _row, 0)

    _accum(accA, di1, e0_ref, True)
    _accum(accA, di2, e0_ref, False)
    _accum(accA, di1, e1_ref, False)
    _accum(accA, di2, e1_ref, False)
    _accum(accA, di1, e2_ref, False)
    _accum(accA, di2, e2_ref, False)

    _accum(accB, ri1, e0_ref, True)
    _accum(accB, ri2, e0_ref, False)
    _accum(accB, ri1, e1_ref, False)
    _accum(accB, ri2, e1_ref, False)
    _accum(accB, ri1, e2_ref, False)
    _accum(accB, ri2, e2_ref, False)

    def _pair(e, _):
        p = accA[e, pl.ds(0, 16)] * accB[e, pl.ds(0, 16)]
        for v in range(1, _D // 16):
            sl = pl.ds(16 * v, 16)
            p = p + accA[e, sl] * accB[e, sl]
        psm[e] = _hsum16(p) * (1.0 / 9.0)
        return 0

    lax.fori_loop(0, _PC, _pair, 0)

    lane = lax.iota(jnp.int32, 16)

    def _pack(g, _):
        accv = jnp.zeros((16,), jnp.float32)
        for jj in range(16):
            accv = jnp.where(lane == jj, psm[g * 16 + jj], accv)
        obuf[pl.ds(g * 16, 16)] = accv
        return 0

    lax.fori_loop(0, _PC // 16, _pack, 0)
    pltpu.sync_copy(obuf, out_ref.at[pl.ds(base, _PC)])


def kernel(diseases, drugs, labels, disease_table, drug_table, Wd, bd, Wr, br,
           g1_idx, g1_vals, g2_idx, g2_vals):
    dd = _gate(disease_table, Wd, bd)
    rr = _gate(drug_table, Wr, br)
    ego = jnp.concatenate([disease_table, drug_table], axis=0)
    ego_g = jnp.concatenate([dd, rr], axis=0)
    e0_flat = jnp.concatenate([ego, ego_g], axis=0)

    g1i = g1_idx.astype(jnp.int32)
    g2i = g2_idx.astype(jnp.int32)
    e1h = g1i.shape[1] // 2
    e2h = g2i.shape[1] // 4
    pa1, pb1 = g1i[0, :e1h], g1i[1, :e1h]
    vf1, vb1 = g1_vals[:e1h], g1_vals[e1h:]
    pa2 = jnp.concatenate([g2i[0, :e2h], g2i[0, 2 * e2h:3 * e2h]])
    pb2 = jnp.concatenate([g2i[1, :e2h], g2i[1, 2 * e2h:3 * e2h]])
    vf2 = jnp.concatenate([g2_vals[:e2h], g2_vals[2 * e2h:3 * e2h]])
    vb2 = jnp.concatenate([g2_vals[e2h:2 * e2h], g2_vals[3 * e2h:]])
    ga = jnp.concatenate([pa1, pa2 + _N])
    gb = jnp.concatenate([pb1, pb2 + _N])
    la = jnp.concatenate([pa1, pa2])
    lb = jnp.concatenate([pb1, pb2])
    vf = jax.lax.bitcast_convert_type(jnp.concatenate([vf1, vf2]), jnp.int32)
    vb = jax.lax.bitcast_convert_type(jnp.concatenate([vb1, vb2]), jnp.int32)
    comb = jnp.concatenate(
        [ga.reshape(-1, _P), gb.reshape(-1, _P), la.reshape(-1, _P),
         lb.reshape(-1, _P), vf.reshape(-1, _P), vb.reshape(-1, _P)],
        axis=1).reshape(-1)

    e1_flat, e2_flat = _prop2(e0_flat, comb)

    dis1 = diseases.astype(jnp.int32)
    dis2 = dis1 + _N
    drg1 = drugs.astype(jnp.int32) + _ND
    drg2 = drg1 + _N
    return _score(e0_flat, e1_flat, e2_flat, dis1, dis2, drg1, drg2)


# speedup vs baseline: 1.0519x; 1.0519x over previous
import functools

import jax
import jax.numpy as jnp
from jax import lax
from jax.experimental import pallas as pl
from jax.experimental.pallas import tpu as pltpu
from jax.experimental.pallas import tpu_sc as plsc

_ND = 4000
_NR = 6000
_N = 10000
_D = 128
_E = 256000
_NC = 2
_NS = 16
_B = 4096

_C = 64
_P = _C // 2
_EPT = _E // _NS
_NCHUNK = _EPT // _C
_NPAIRIT = _NCHUNK // 2
_IW = 6 * _P

_RPT = 624
_RC = 104
_RTAIL = _N - _NS * _RPT


def _gate_body(dt_ref, rt_ref, wd_ref, bd_ref, wr_ref, br_ref, out_ref):
    d = dt_ref[...]
    r = rt_ref[...]
    out_ref[0:_ND, :] = d
    out_ref[_ND:_N, :] = r
    zd = jnp.dot(d, wd_ref[...], preferred_element_type=jnp.float32) + bd_ref[...]
    out_ref[_N:_N + _ND, :] = d * jax.nn.sigmoid(zd)
    zr = jnp.dot(r, wr_ref[...], preferred_element_type=jnp.float32) + br_ref[...]
    out_ref[_N + _ND:, :] = r * jax.nn.sigmoid(zr)


def _gate_all(dt, rt, Wd, bd, Wr, br):
    return pl.pallas_call(
        _gate_body,
        out_shape=jax.ShapeDtypeStruct((2 * _N, _D), jnp.float32),
    )(dt, rt, Wd, bd, Wr, br)


def _prop_mesh():
    return plsc.VectorSubcoreMesh(core_axis_name="c", subcore_axis_name="s")


def _hsum16(p):
    return jnp.sum(p)


@functools.partial(
    pl.kernel,
    out_type=[
        jax.ShapeDtypeStruct((2 * _N, _D), jnp.float32),
        jax.ShapeDtypeStruct((2 * _N, _D), jnp.float32),
    ],
    mesh=_prop_mesh(),
    compiler_params=pltpu.CompilerParams(needs_layout_passes=False),
    scratch_types=[
        pltpu.VMEM_SHARED((_N, _D), jnp.float32),
        pltpu.VMEM((_C, _D), jnp.float32),
        pltpu.VMEM((_C, _D), jnp.float32),
        pltpu.VMEM((_C, _D), jnp.float32),
        pltpu.VMEM((_C, _D), jnp.float32),
        pltpu.VMEM((_RC, _D), jnp.float32),
        pltpu.VMEM((_IW,), jnp.int32),
        pltpu.VMEM((_IW,), jnp.int32),
        pltpu.VMEM((_C,), jnp.int32),
        pltpu.VMEM((_C,), jnp.int32),
        pltpu.SMEM((_P,), jnp.float32),
        pltpu.SMEM((_C,), jnp.float32),
        pltpu.SemaphoreType.DMA,
        pltpu.SemaphoreType.DMA,
        pltpu.SemaphoreType.DMA,
        pltpu.SemaphoreType.DMA,
        pltpu.SemaphoreType.DMA,
        pltpu.SemaphoreType.DMA,
    ],
)
def _prop2(e0_ref, comb_ref, e1_ref, e2_ref,
           acc, gbufA, gbufB, wbufA, wbufB, dbuf, idxA, idxB, isccA, isccB,
           dsm, msm, gsA, gsB, ssA, ssB, isA, isB):
    c = lax.axis_index("c")
    s = lax.axis_index("s")
    cbase = c * (_E // _C) + s * _NCHUNK

    def _zero_dbuf():
        def _zrow(i, _):
            for v in range(_D // 16):
                dbuf[i, pl.ds(16 * v, 16)] = jnp.zeros((16,), jnp.float32)
            return 0
        lax.fori_loop(0, _RC, _zrow, 0)

    def _acc_zero():
        for r in range(_RPT // _RC):
            pltpu.sync_copy(dbuf,
                            acc.at[pl.ds(s * _RPT + r * _RC, _RC), :])

        @pl.when(s == 0)
        def _():
            pltpu.sync_copy(dbuf.at[pl.ds(0, _RTAIL), :],
                            acc.at[pl.ds(_NS * _RPT, _RTAIL), :])

    _zero_dbuf()
    _acc_zero()
    plsc.subcore_barrier()

    def _stage_iscc(idx, iscc):
        for v in range(_C // 16):
            iscc[pl.ds(16 * v, 16)] = idx[pl.ds(2 * _P + 16 * v, 16)]

    def _compute(gbuf, idx, wbuf):
        def _dot(e, _):
            p01 = (gbuf[e, pl.ds(0, 16)] * gbuf[_P + e, pl.ds(0, 16)]
                   + gbuf[e, pl.ds(16, 16)] * gbuf[_P + e, pl.ds(16, 16)])
            p23 = (gbuf[e, pl.ds(32, 16)] * gbuf[_P + e, pl.ds(32, 16)]
                   + gbuf[e, pl.ds(48, 16)] * gbuf[_P + e, pl.ds(48, 16)])
            p45 = (gbuf[e, pl.ds(64, 16)] * gbuf[_P + e, pl.ds(64, 16)]
                   + gbuf[e, pl.ds(80, 16)] * gbuf[_P + e, pl.ds(80, 16)])
            p67 = (gbuf[e, pl.ds(96, 16)] * gbuf[_P + e, pl.ds(96, 16)]
                   + gbuf[e, pl.ds(112, 16)] * gbuf[_P + e, pl.ds(112, 16)])
            p = (p01 + p23) + (p45 + p67)
            dsm[e] = 2.0 - _hsum16(p) * (1.0 / 64.0)
            return 0

        lax.fori_loop(0, _P, _dot, 0)

        def _mul(g, _):
            vf = plsc.bitcast(idx[pl.ds(4 * _P + 16 * g, 16)], jnp.float32)
            vb = plsc.bitcast(idx[pl.ds(5 * _P + 16 * g, 16)], jnp.float32)
            for jj in range(16):
                e = g * 16 + jj
                d = dsm[e]
                msm[e] = d * vf[jj]
                msm[_P + e] = d * vb[jj]
            return 0

        lax.fori_loop(0, _P // 16, _mul, 0)

        def _scale(e, _):
            mf = msm[e]
            mb = msm[_P + e]
            for v in range(8):
                sl = pl.ds(16 * v, 16)
                ra = gbuf[e, sl]
                rb = gbuf[_P + e, sl]
                wbuf[e, sl] = rb * mf
                wbuf[_P + e, sl] = ra * mb
            return 0

        lax.fori_loop(0, _P, _scale, 0)

    def _fire_gather(src_ref, idx, gbuf, sem):
        pltpu.async_copy(src_ref.at[idx.at[pl.ds(0, _C)]], gbuf, sem)

    def _wait_gather(src_ref, idx, gbuf, sem):
        pltpu.make_async_copy(
            src_ref.at[idx.at[pl.ds(0, _C)]], gbuf, sem).wait()

    def _fire_scatter(wbuf, iscc, sem):
        pltpu.async_copy(wbuf, acc.at[iscc], sem, add=True)

    def _wait_scatter(wbuf, iscc, sem):
        pltpu.make_async_copy(wbuf, acc.at[iscc], sem).wait()

    def _fire_idx(blk, idx, sem):
        pltpu.async_copy(comb_ref.at[pl.ds(blk * _IW, _IW)], idx, sem)

    def _wait_idx(idx, sem):
        pltpu.make_async_copy(comb_ref.at[pl.ds(0, _IW)], idx, sem).wait()

    def _layer(src_ref, dst_ref, rezero):
        pltpu.sync_copy(comb_ref.at[pl.ds(cbase * _IW, _IW)], idxA)
        pltpu.sync_copy(comb_ref.at[pl.ds((cbase + 1) * _IW, _IW)], idxB)
        _fire_gather(src_ref, idxA, gbufA, gsA)

        def _pair(q, _):
            j0 = 2 * q

            _wait_gather(src_ref, idxA, gbufA, gsA)

            @pl.when(q > 0)
            def _():
                _wait_scatter(wbufA, isccA, ssA)
                _wait_idx(idxB, isB)

            _stage_iscc(idxA, isccA)
            _fire_gather(src_ref, idxB, gbufB, gsB)
            _compute(gbufA, idxA, wbufA)
            _fire_scatter(wbufA, isccA, ssA)

            @pl.when(j0 + 2 < _NCHUNK)
            def _():
                _fire_idx(cbase + j0 + 2, idxA, isA)

            _wait_gather(src_ref, idxB, gbufB, gsB)

            @pl.when(q > 0)
            def _():
                _wait_scatter(wbufB, isccB, ssB)

            _stage_iscc(idxB, isccB)

            @pl.when(j0 + 2 < _NCHUNK)
            def _():
                _wait_idx(idxA, isA)
                _fire_gather(src_ref, idxA, gbufA, gsA)

            _compute(gbufB, idxB, wbufB)
            _fire_scatter(wbufB, isccB, ssB)

            @pl.when(j0 + 3 < _NCHUNK)
            def _():
                _fire_idx(cbase + j0 + 3, idxB, isB)

            return 0

        lax.fori_loop(0, _NPAIRIT, _pair, 0)
        _wait_scatter(wbufA, isccA, ssA)
        _wait_scatter(wbufB, isccB, ssB)
        plsc.subcore_barrier()

        def _dump(row0, nrows):
            pltpu.sync_copy(acc.at[pl.ds(row0, nrows), :],
                            dbuf.at[pl.ds(0, nrows), :])
            pltpu.sync_copy(dbuf.at[pl.ds(0, nrows), :],
                            dst_ref.at[pl.ds(c * _N + row0, nrows), :])

        for r in range(_RPT // _RC):
            _dump(s * _RPT + r * _RC, _RC)

        @pl.when(s == 0)
        def _():
            _dump(_NS * _RPT, _RTAIL)

        if rezero:
            _zero_dbuf()
            _acc_zero()
        plsc.subcore_barrier()

    _layer(e0_ref, e1_ref, True)
    _layer(e1_ref, e2_ref, False)


_PC = _B // (_NC * _NS)


@functools.partial(
    pl.kernel,
    out_type=jax.ShapeDtypeStruct((_B,), jnp.float32),
    mesh=_prop_mesh(),
    compiler_params=pltpu.CompilerParams(needs_layout_passes=False),
    scratch_types=[
        pltpu.VMEM((_PC,), jnp.int32),
        pltpu.VMEM((_PC,), jnp.int32),
        pltpu.VMEM((_PC,), jnp.int32),
        pltpu.VMEM((_PC,), jnp.int32),
        pltpu.VMEM((_PC, _D), jnp.float32),
        pltpu.VMEM((_PC, _D), jnp.float32),
        pltpu.VMEM((_PC, _D), jnp.float32),
        pltpu.VMEM((_PC, _D), jnp.float32),
        pltpu.VMEM((_PC,), jnp.float32),
        pltpu.SMEM((_PC,), jnp.float32),
        pltpu.SemaphoreType.DMA,
        pltpu.SemaphoreType.DMA,
    ],
)
def _score(e0_ref, e1_ref, e2_ref, d1_ref, d2_ref, r1_ref, r2_ref, out_ref,
           di1, di2, ri1, ri2, accA, accB, tmpA, tmpB, obuf, psm, semA, semB):
    c = lax.axis_index("c")
    s = lax.axis_index("s")
    wid = s * _NC + c
    base = wid * _PC

    pltpu.sync_copy(d1_ref.at[pl.ds(base, _PC)], di1)
    pltpu.sync_copy(d2_ref.at[pl.ds(base, _PC)], di2)
    pltpu.sync_copy(r1_ref.at[pl.ds(base, _PC)], ri1)
    pltpu.sync_copy(r2_ref.at[pl.ds(base, _PC)], ri2)

    seq = [
        (accA, di1, e0_ref, True), (accA, di2, e0_ref, False),
        (accA, di1, e1_ref, False), (accA, di2, e1_ref, False),
        (accA, di1, e2_ref, False), (accA, di2, e2_ref, False),
        (accB, ri1, e0_ref, True), (accB, ri2, e0_ref, False),
        (accB, ri1, e1_ref, False), (accB, ri2, e1_ref, False),
        (accB, ri1, e2_ref, False), (accB, ri2, e2_ref, False),
    ]
    banks = [(tmpA, semA), (tmpB, semB)]
    pltpu.async_copy(seq[0][2].at[seq[0][1]], tmpA, semA)
    for k, (dst, idx_ref, table_ref, first) in enumerate(seq):
        tmp, sem = banks[k % 2]
        if k + 1 < len(seq):
            ndst, nidx, ntab, _ = seq[k + 1]
            ntmp, nsem = banks[(k + 1) % 2]
            pltpu.async_copy(ntab.at[nidx], ntmp, nsem)
        pltpu.make_async_copy(table_ref.at[idx_ref], tmp, sem).wait()

        def _row(e, _, dst=dst, tmp=tmp, first=first):
            for v in range(_D // 16):
                sl = pl.ds(16 * v, 16)
                if first:
                    dst[e, sl] = tmp[e, sl]
                else:
                    dst[e, sl] = dst[e, sl] + tmp[e, sl]
            return 0

        lax.fori_loop(0, _PC, _row, 0)

    def _pair(e, _):
        p = accA[e, pl.ds(0, 16)] * accB[e, pl.ds(0, 16)]
        for v in range(1, _D // 16):
            sl = pl.ds(16 * v, 16)
            p = p + accA[e, sl] * accB[e, sl]
        psm[e] = _hsum16(p) * (1.0 / 9.0)
        return 0

    lax.fori_loop(0, _PC, _pair, 0)

    lane = lax.iota(jnp.int32, 16)

    def _pack(g, _):
        accv = jnp.zeros((16,), jnp.float32)
        for jj in range(16):
            accv = jnp.where(lane == jj, psm[g * 16 + jj], accv)
        obuf[pl.ds(g * 16, 16)] = accv
        return 0

    lax.fori_loop(0, _PC // 16, _pack, 0)
    pltpu.sync_copy(obuf, out_ref.at[pl.ds(base, _PC)])


def kernel(diseases, drugs, labels, disease_table, drug_table, Wd, bd, Wr, br,
           g1_idx, g1_vals, g2_idx, g2_vals):
    e0_flat = _gate_all(disease_table, drug_table, Wd, bd, Wr, br)

    g1i = g1_idx.astype(jnp.int32)
    g2i = g2_idx.astype(jnp.int32)
    e1h = g1i.shape[1] // 2
    e2h = g2i.shape[1] // 4
    pa1, pb1 = g1i[0, :e1h], g1i[1, :e1h]
    vf1, vb1 = g1_vals[:e1h], g1_vals[e1h:]
    pa2 = jnp.concatenate([g2i[0, :e2h], g2i[0, 2 * e2h:3 * e2h]])
    pb2 = jnp.concatenate([g2i[1, :e2h], g2i[1, 2 * e2h:3 * e2h]])
    vf2 = jnp.concatenate([g2_vals[:e2h], g2_vals[2 * e2h:3 * e2h]])
    vb2 = jnp.concatenate([g2_vals[e2h:2 * e2h], g2_vals[3 * e2h:]])
    ga = jnp.concatenate([pa1, pa2 + _N])
    gb = jnp.concatenate([pb1, pb2 + _N])
    la = jnp.concatenate([pa1, pa2])
    lb = jnp.concatenate([pb1, pb2])
    vf = jax.lax.bitcast_convert_type(jnp.concatenate([vf1, vf2]), jnp.int32)
    vb = jax.lax.bitcast_convert_type(jnp.concatenate([vb1, vb2]), jnp.int32)
    comb = jnp.concatenate(
        [ga.reshape(-1, _P), gb.reshape(-1, _P), la.reshape(-1, _P),
         lb.reshape(-1, _P), vf.reshape(-1, _P), vb.reshape(-1, _P)],
        axis=1).reshape(-1)

    e1_flat, e2_flat = _prop2(e0_flat, comb)

    dis1 = diseases.astype(jnp.int32)
    dis2 = dis1 + _N
    drg1 = drugs.astype(jnp.int32) + _ND
    drg2 = drg1 + _N
    return _score(e0_flat, e1_flat, e2_flat, dis1, dis2, drg1, drg2)
